# Initial kernel scaffold; baseline (speedup 1.0000x reference)
import jax
import jax.numpy as jnp
from jax.experimental import pallas as pl


def kernel(x, edge_index, W1, a_src1, a_dst1, b1, W2, a_src2, a_dst2, b2, W3, a_src3, a_dst3, b3):
    raise NotImplementedError("write your pallas kernel here")


# speedup vs baseline: 1.0700x; 1.0700x over previous
import jax
import jax.numpy as jnp
from jax.experimental import pallas as pl

N = 10000
HEADS = (7, 7, 1)
CH = (16, 16, 40)


def _gat_layer(x, src, dst, W, a_src, a_dst, b, heads, ch):
    n = x.shape[0]
    h = (x @ W).reshape(n, heads, ch)
    asrc = (h * a_src[None]).sum(-1)
    adst = (h * a_dst[None]).sum(-1)
    bound = asrc.max(0) + adst.max(0)
    bound = jnp.where(bound > 0, bound, 0.2 * bound)
    e = asrc[src] + adst[dst]
    e = jnp.where(e > 0, e, 0.2 * e)
    ex = jnp.exp(e - bound[None, :])
    denom = jax.ops.segment_sum(ex, dst, num_segments=n)
    alpha = ex / (denom[dst] + 1e-16)
    out = jax.ops.segment_sum(h[src] * alpha[:, :, None], dst, num_segments=n)
    return out.reshape(n, heads * ch) + b


def _logsoftmax_kernel(x_ref, o_ref):
    x = x_ref[...]
    m = jnp.max(x, axis=1, keepdims=True)
    s = jnp.log(jnp.sum(jnp.exp(x - m), axis=1, keepdims=True))
    o_ref[...] = x - m - s


def kernel(x, edge_index, W1, a_src1, a_dst1, b1, W2, a_src2, a_dst2, b2, W3, a_src3, a_dst3, b3):
    src = jnp.concatenate([edge_index[0], jnp.arange(N, dtype=edge_index.dtype)]).astype(jnp.int32)
    dst = jnp.concatenate([edge_index[1], jnp.arange(N, dtype=edge_index.dtype)]).astype(jnp.int32)
    h = _gat_layer(x, src, dst, W1, a_src1, a_dst1, b1, HEADS[0], CH[0])
    h = jax.nn.relu(h)
    h = _gat_layer(h, src, dst, W2, a_src2, a_dst2, b2, HEADS[1], CH[1])
    h = jax.nn.relu(h)
    h = _gat_layer(h, src, dst, W3, a_src3, a_dst3, b3, HEADS[2], CH[2])
    return pl.pallas_call(
        _logsoftmax_kernel,
        out_shape=jax.ShapeDtypeStruct((N, 40), jnp.float32),
        grid=(10,),
        in_specs=[pl.BlockSpec((N // 10, 40), lambda i: (i, 0))],
        out_specs=pl.BlockSpec((N // 10, 40), lambda i: (i, 0)),
    )(h)


# speedup vs baseline: 34.4751x; 32.2191x over previous
import functools

import jax
import jax.numpy as jnp
from jax import lax
from jax.experimental import pallas as pl
from jax.experimental.pallas import tpu as pltpu
from jax.experimental.pallas import tpu_sc as plsc

N = 10000
E = 320000
NC, NS = 2, 16
NW = NC * NS
C = 128
N_PAD = 10112
RPS = N_PAD // NS
E2 = 331776
NCH = E2 // (NW * C)
BN = 1000


def _lrelu(v):
    return jnp.where(v > 0, v, 0.2 * v)


def _dense_body(first, dout, dpad, *refs):
    i = pl.program_id(0)
    if first:
        x_ref, w_ref, ab_ref, h_ref, s_ref, d_ref, bm_ref = refs
        x = x_ref[...]
    else:
        p0_ref, p1_ref, b_ref, w_ref, ab_ref, h_ref, s_ref, d_ref, bm_ref = refs
        x = jnp.maximum(p0_ref[...] + p1_ref[...] + b_ref[...], 0.0)
    h = jnp.dot(x, w_ref[...], preferred_element_type=jnp.float32)
    al = jnp.dot(h, ab_ref[...], preferred_element_type=jnp.float32)
    if dpad > dout:
        h = jnp.concatenate([h, jnp.zeros((h.shape[0], dpad - dout), jnp.float32)], 1)
    h_ref[...] = h
    s_ref[...] = al[:, :16]
    d_ref[...] = al[:, 16:32]
    red = jnp.max(jnp.concatenate([al, jnp.full((al.shape[0], 96), -30.0)], 1),
                  axis=0, keepdims=True)
    red = jnp.broadcast_to(red, (8, 128))

    @pl.when(i == 0)
    def _():
        bm_ref[...] = jnp.full((8, 128), -30.0, jnp.float32)

    bm_ref[...] = jnp.maximum(bm_ref[...], red)


def _dense(first, din, dout, dpad, args):
    body = functools.partial(_dense_body, first, dout, dpad)
    n_in = 4 if first else 6
    in_specs = ([pl.BlockSpec((BN, din), lambda i: (i, 0))] if first else [
        pl.BlockSpec((BN, din), lambda i: (i, 0)),
        pl.BlockSpec((BN, din), lambda i: (i, 0)),
        pl.BlockSpec((1, din), lambda i: (0, 0)),
    ])
    in_specs += [
        pl.BlockSpec((din, dout), lambda i: (0, 0)),
        pl.BlockSpec((dout, 32), lambda i: (0, 0)),
    ]
    return pl.pallas_call(
        body,
        grid=(N // BN,),
        in_specs=in_specs,
        out_specs=[
            pl.BlockSpec((BN, dpad), lambda i: (i, 0)),
            pl.BlockSpec((BN, 16), lambda i: (i, 0)),
            pl.BlockSpec((BN, 16), lambda i: (i, 0)),
            pl.BlockSpec((8, 128), lambda i: (0, 0)),
        ],
        out_shape=[
            jax.ShapeDtypeStruct((N, dpad), jnp.float32),
            jax.ShapeDtypeStruct((N, 16), jnp.float32),
            jax.ShapeDtypeStruct((N, 16), jnp.float32),
            jax.ShapeDtypeStruct((8, 128), jnp.float32),
        ],
    )(*args)


def _mid_body(ad_ref, d0_ref, d1_ref, o_ref):
    dinv = 1.0 / (d0_ref[...] + d1_ref[...] + 1e-16)
    o_ref[...] = jnp.concatenate([ad_ref[...][:, :8], dinv[:, :8]], 1)


def _mid(ad, d0, d1):
    return pl.pallas_call(
        _mid_body,
        grid=(N // BN,),
        in_specs=[pl.BlockSpec((BN, 16), lambda i: (i, 0))] * 3,
        out_specs=pl.BlockSpec((BN, 16), lambda i: (i, 0)),
        out_shape=jax.ShapeDtypeStruct((N, 16), jnp.float32),
    )(ad, d0, d1)


def _final_body(p0_ref, p1_ref, b_ref, o_ref):
    v = p0_ref[...][:, :40] + p1_ref[...][:, :40] + b_ref[...]
    m = jnp.max(v, axis=1, keepdims=True)
    s = jnp.log(jnp.sum(jnp.exp(v - m), axis=1, keepdims=True))
    o_ref[...] = v - m - s


def _final(p0, p1, b):
    return pl.pallas_call(
        _final_body,
        grid=(N // BN,),
        in_specs=[
            pl.BlockSpec((BN, 48), lambda i: (i, 0)),
            pl.BlockSpec((BN, 48), lambda i: (i, 0)),
            pl.BlockSpec((1, 40), lambda i: (0, 0)),
        ],
        out_specs=pl.BlockSpec((BN, 40), lambda i: (i, 0)),
        out_shape=jax.ShapeDtypeStruct((N, 40), jnp.float32),
    )(p0, p1, b)


_MESH = plsc.VectorSubcoreMesh(
    core_axis_name="c", subcore_axis_name="s", num_cores=NC, num_subcores=NS)


def _bvec(b0_ref, b1_ref):
    return _lrelu(b0_ref[...] + b1_ref[...])


def _dg(v, idx):
    dn = lax.GatherDimensionNumbers(
        offset_dims=(), collapsed_slice_dims=(0,), start_index_map=(0,))
    return lax.gather(v, idx[:, None], dn, (1,),
                      mode=lax.GatherScatterMode.PROMISE_IN_BOUNDS)


@functools.partial(
    pl.kernel,
    out_type=jax.ShapeDtypeStruct((NC, N_PAD, 16), jnp.float32),
    mesh=_MESH,
    compiler_params=pltpu.CompilerParams(use_tc_tiling_on_sc=False),
    scratch_types=[
        pltpu.VMEM((C,), jnp.int32),
        pltpu.VMEM((C,), jnp.int32),
        pltpu.VMEM((C, 16), jnp.float32),
        pltpu.VMEM((C, 16), jnp.float32),
        pltpu.VMEM((C, 16), jnp.float32),
        pltpu.VMEM((16,), jnp.float32),
        pltpu.VMEM((16,), jnp.float32),
        pltpu.VMEM_SHARED((N_PAD, 16), jnp.float32),
        pltpu.SemaphoreType.DMA,
        pltpu.SemaphoreType.DMA,
    ],
)
def _pass_a(src_h, dst_h, as_h, ad_h, bm_h, z_h, out_h,
            srcv, dstv, asr, adr, exb, b0v, b1v, dsh, sem1, sem2):
    cid = lax.axis_index("c")
    sid = lax.axis_index("s")
    wid = cid * NS + sid
    row0 = pl.multiple_of(sid * RPS, 8)
    pltpu.sync_copy(z_h.at[pl.ds(row0, RPS)], dsh.at[pl.ds(row0, RPS)])
    pltpu.sync_copy(bm_h.at[pl.ds(0, 16)], b0v)
    pltpu.sync_copy(bm_h.at[pl.ds(16, 16)], b1v)
    plsc.subcore_barrier()
    base = wid * (NCH * C)

    def chunk(g, carry):
        off = pl.multiple_of(base + g * C, 8)
        pltpu.sync_copy(src_h.at[pl.ds(off, C)], srcv)
        pltpu.sync_copy(dst_h.at[pl.ds(off, C)], dstv)
        cp1 = pltpu.async_copy(as_h.at[srcv], asr, sem1)
        cp2 = pltpu.async_copy(ad_h.at[dstv], adr, sem2)
        cp1.wait()
        cp2.wait()
        bvec = _bvec(b0v, b1v)

        def edge(e, c2):
            s = asr[e] + adr[e]
            exb[e] = jnp.exp(_lrelu(s) - bvec)
            return c2

        lax.fori_loop(0, C, edge, 0)
        pltpu.sync_copy(exb, dsh.at[dstv], add=True)
        return carry

    lax.fori_loop(0, NCH, chunk, 0)
    plsc.subcore_barrier()
    pltpu.sync_copy(dsh.at[pl.ds(row0, RPS)],
                    out_h.at[cid, pl.ds(row0, RPS)])


def _make_pass_b(dpad, heads):
    nh_vec = dpad // 16

    @functools.partial(
        pl.kernel,
        out_type=jax.ShapeDtypeStruct((NC, N_PAD, dpad), jnp.float32),
        mesh=_MESH,
        compiler_params=pltpu.CompilerParams(use_tc_tiling_on_sc=False),
        scratch_types=[
            pltpu.VMEM((C,), jnp.int32),
            pltpu.VMEM((C,), jnp.int32),
            pltpu.VMEM((C, 16), jnp.float32),
            pltpu.VMEM((C, 16), jnp.float32),
            pltpu.VMEM((C, dpad), jnp.float32),
            pltpu.VMEM((C, dpad), jnp.float32),
            pltpu.VMEM((16,), jnp.float32),
            pltpu.VMEM((16,), jnp.float32),
            pltpu.VMEM_SHARED((N_PAD, dpad), jnp.float32),
            pltpu.SemaphoreType.DMA,
            pltpu.SemaphoreType.DMA,
            pltpu.SemaphoreType.DMA,
        ],
    )
    def _pass_b(src_h, dst_h, as_h, ad2_h, bm_h, h_h, z_h, out_h,
                srcv, dstv, asr, adr, hr, scb, b0v, b1v, osh,
                sem1, sem2, sem3):
        cid = lax.axis_index("c")
        sid = lax.axis_index("s")
        wid = cid * NS + sid
        row0 = pl.multiple_of(sid * RPS, 8)
        pltpu.sync_copy(z_h.at[pl.ds(row0, RPS)],
                        osh.at[pl.ds(row0, RPS)])
        pltpu.sync_copy(bm_h.at[pl.ds(0, 16)], b0v)
        pltpu.sync_copy(bm_h.at[pl.ds(16, 16)], b1v)
        plsc.subcore_barrier()
        base = wid * (NCH * C)
        lanes = jnp.arange(16, dtype=jnp.int32)
        sh8 = (lanes & 7) + 8

        def chunk(g, carry):
            off = pl.multiple_of(base + g * C, 8)
            pltpu.sync_copy(src_h.at[pl.ds(off, C)], srcv)
            pltpu.sync_copy(dst_h.at[pl.ds(off, C)], dstv)
            cp1 = pltpu.async_copy(as_h.at[srcv], asr, sem1)
            cp2 = pltpu.async_copy(ad2_h.at[dstv], adr, sem2)
            cp3 = pltpu.async_copy(h_h.at[srcv], hr, sem3)
            cp1.wait()
            cp2.wait()
            cp3.wait()
            bvec = _bvec(b0v, b1v)

            def edge(e, c2):
                s = asr[e] + adr[e]
                ex = jnp.exp(_lrelu(s) - bvec)
                alpha = ex * _dg(s, sh8)
                if heads == 1:
                    bk = _dg(alpha, jnp.zeros((16,), jnp.int32))
                    for k in range(nh_vec):
                        scb[e, pl.ds(16 * k, 16)] = bk * hr[e, pl.ds(16 * k, 16)]
                else:
                    for k in range(nh_vec):
                        bk = _dg(alpha, jnp.full((16,), k, jnp.int32))
                        scb[e, pl.ds(16 * k, 16)] = bk * hr[e, pl.ds(16 * k, 16)]
                return c2

            lax.fori_loop(0, C, edge, 0)
            pltpu.sync_copy(scb, osh.at[dstv], add=True)
            return carry

        lax.fori_loop(0, NCH, chunk, 0)
        plsc.subcore_barrier()
        pltpu.sync_copy(osh.at[pl.ds(row0, RPS)],
                        out_h.at[cid, pl.ds(row0, RPS)])

    return _pass_b


_pass_b112 = _make_pass_b(112, 7)
_pass_b48 = _make_pass_b(48, 1)


def _blockdiag(a):
    heads, ch = a.shape
    m = jnp.zeros((heads * ch, 16), a.dtype)
    r = jnp.arange(heads * ch)
    return m.at[r, r // ch].set(a.reshape(-1))


def _layer(first, src, dst, xargs, W, a_src, a_dst, dout, dpad, heads, pass_b, z16, zD):
    din = W.shape[0]
    ab = jnp.concatenate([_blockdiag(a_src), _blockdiag(a_dst)], 1)
    h, AS, AD, bm = _dense(first, din, dout, dpad, list(xargs) + [W, ab])
    bmf = bm.reshape(-1)
    dp = _pass_a(src, dst, AS, AD, bmf, z16)
    AD2 = _mid(AD, dp[0, :N], dp[1, :N])
    p = pass_b(src, dst, AS, AD2, bmf, h, zD)
    return p[0, :N], p[1, :N]


def kernel(x, edge_index, W1, a_src1, a_dst1, b1, W2, a_src2, a_dst2, b2,
           W3, a_src3, a_dst3, b3):
    ei = edge_index.astype(jnp.int32)
    loop = jnp.arange(N, dtype=jnp.int32)
    padi = jnp.arange(E2 - N - E, dtype=jnp.int32)
    src = jnp.concatenate([ei[0], loop, padi % 16])
    dst = jnp.concatenate([ei[1], loop, N + (padi % 16)])
    z16 = jnp.zeros((N_PAD, 16), jnp.float32)
    z112 = jnp.zeros((N_PAD, 112), jnp.float32)
    z48 = jnp.zeros((N_PAD, 48), jnp.float32)

    p0, p1 = _layer(True, src, dst, (x,), W1, a_src1, a_dst1,
                    112, 112, 7, _pass_b112, z16, z112)
    p0, p1 = _layer(False, src, dst, (p0, p1, b1.reshape(1, -1)), W2,
                    a_src2, a_dst2, 112, 112, 7, _pass_b112, z16, z112)
    p0, p1 = _layer(False, src, dst, (p0, p1, b2.reshape(1, -1)), W3,
                    a_src3, a_dst3, 40, 48, 1, _pass_b48, z16, z48)
    return _final(p0, p1, b3.reshape(1, -1))
